# Initial kernel scaffold; baseline (speedup 1.0000x reference)
import jax
import jax.numpy as jnp
from jax.experimental import pallas as pl


def kernel(x, edge_index, batch, W1, att_src1, att_dst1, b1, W2, att_src2, att_dst2, b2):
    raise NotImplementedError("write your pallas kernel here")


# speedup vs baseline: 38.1931x; 38.1931x over previous
import functools

import jax
import jax.numpy as jnp
from jax import lax
from jax.experimental import pallas as pl
from jax.experimental.pallas import tpu as pltpu
from jax.experimental.pallas import tpu_sc as plsc

N = 10000
E = 320000
F_IN = 128
H = 8
C = 8
HC = 64
OUT = 10
G = 128

NP = 10240
EP = 32 * NP
TILE_ROWS = NP // 16
NBATCH = (EP // 32) // 128


def _k1_body(x_ref, w_ref, a_ref, h_ref, asd_ref):
    h = jnp.dot(x_ref[...], w_ref[...], preferred_element_type=jnp.float32)
    h_ref[...] = h
    asd_ref[...] = jnp.dot(h, a_ref[...], preferred_element_type=jnp.float32)


def _k1(x_pad, W1, A1):
    B = 1280
    return pl.pallas_call(
        _k1_body,
        grid=(NP // B,),
        in_specs=[
            pl.BlockSpec((B, F_IN), lambda i: (i, 0)),
            pl.BlockSpec((F_IN, HC), lambda i: (0, 0)),
            pl.BlockSpec((HC, 16), lambda i: (0, 0)),
        ],
        out_specs=[
            pl.BlockSpec((B, HC), lambda i: (i, 0)),
            pl.BlockSpec((B, 16), lambda i: (i, 0)),
        ],
        out_shape=[
            jax.ShapeDtypeStruct((NP, HC), jnp.float32),
            jax.ShapeDtypeStruct((NP, 16), jnp.float32),
        ],
    )(x_pad, W1, A1)


def _k5_body(oa_ref, ob_ref, b1_ref, w2_ref, a2_ref, h2_ref, asd_ref):
    o = oa_ref[...] + ob_ref[...] + b1_ref[...]
    x2 = jnp.where(o > 0.0, o, jnp.exp(jnp.minimum(o, 0.0)) - 1.0)
    h2 = jnp.dot(x2, w2_ref[...], preferred_element_type=jnp.float32)
    h2_ref[...] = h2
    asd_ref[...] = jnp.dot(h2, a2_ref[...], preferred_element_type=jnp.float32)


def _k5(outA, outB, b1_2d, W2pad, A2):
    B = 1280
    return pl.pallas_call(
        _k5_body,
        grid=(NP // B,),
        in_specs=[
            pl.BlockSpec((B, HC), lambda i: (i, 0)),
            pl.BlockSpec((B, HC), lambda i: (i, 0)),
            pl.BlockSpec((1, HC), lambda i: (0, 0)),
            pl.BlockSpec((HC, 16), lambda i: (0, 0)),
            pl.BlockSpec((16, 16), lambda i: (0, 0)),
        ],
        out_specs=[
            pl.BlockSpec((B, 16), lambda i: (i, 0)),
            pl.BlockSpec((B, 16), lambda i: (i, 0)),
        ],
        out_shape=[
            jax.ShapeDtypeStruct((NP, 16), jnp.float32),
            jax.ShapeDtypeStruct((NP, 16), jnp.float32),
        ],
    )(outA, outB, b1_2d, W2pad, A2)


def _k8_body(oa_ref, ob_ref, b3_ref, b2_ref, out_ref, acc_ref, cnt_ref):
    i = pl.program_id(0)

    @pl.when(i == 0)
    def _():
        acc_ref[...] = jnp.zeros_like(acc_ref)
        cnt_ref[...] = jnp.zeros_like(cnt_ref)

    rows = oa_ref[...] + ob_ref[...]
    ids = b3_ref[0]
    gids = lax.broadcasted_iota(jnp.int32, (G, ids.shape[1]), 0)
    mask = (ids == gids).astype(jnp.float32)
    acc_ref[...] += jnp.dot(mask, rows, preferred_element_type=jnp.float32)
    cnt_ref[...] += jnp.sum(mask, axis=1, keepdims=True)

    @pl.when(i == pl.num_programs(0) - 1)
    def _():
        cnt = cnt_ref[...]
        out_ref[...] = (acc_ref[...] + cnt * b2_ref[...]) / jnp.maximum(cnt, 1.0)


def _k8(o2A, o2B, batch3, b2_2d):
    B = 1000
    return pl.pallas_call(
        _k8_body,
        grid=(N // B,),
        in_specs=[
            pl.BlockSpec((B, 16), lambda i: (i, 0)),
            pl.BlockSpec((B, 16), lambda i: (i, 0)),
            pl.BlockSpec((1, 1, B), lambda i: (i, 0, 0)),
            pl.BlockSpec((1, 16), lambda i: (0, 0)),
        ],
        out_specs=pl.BlockSpec((G, 16), lambda i: (0, 0)),
        out_shape=jax.ShapeDtypeStruct((G, 16), jnp.float32),
        scratch_shapes=[
            pltpu.VMEM((G, 16), jnp.float32),
            pltpu.VMEM((G, 1), jnp.float32),
        ],
    )(o2A, o2B, batch3, b2_2d)


_MESH = dict(core_axis_name="c", subcore_axis_name="s")


def _sc_phase1_body(asd_hbm, src_hbm, dst_hbm, exbuf_hbm, denA_hbm, denB_hbm,
                    sidx, didx, srows, drows, exrows, asd_sh, den_sh,
                    sem_s, sem_d):
    c = lax.axis_index("c")
    s = lax.axis_index("s")
    wid = c * 16 + s
    base = wid * (EP // 32)
    rbase = s * TILE_ROWS
    rows = pl.ds(rbase, TILE_ROWS)

    pltpu.sync_copy(asd_hbm.at[rows], asd_sh.at[rows])

    def _zero(i, _):
        exrows[i] = jnp.zeros((16,), jnp.float32)
        return _

    lax.fori_loop(0, 128, _zero, None, unroll=8)

    def _zcp(j, _):
        pltpu.sync_copy(exrows, den_sh.at[pl.ds(rbase + j * 128, 128)])
        return _

    lax.fori_loop(0, TILE_ROWS // 128, _zcp, None)
    plsc.subcore_barrier()

    lane = lax.broadcasted_iota(jnp.int32, (16,), 0)
    shift = 8 + (lane % 8)

    def _batch(b, _):
        off = base + b * 128
        pltpu.sync_copy(src_hbm.at[pl.ds(off, 128)], sidx)
        pltpu.sync_copy(dst_hbm.at[pl.ds(off, 128)], didx.at[0])
        pltpu.async_copy(asd_sh.at[sidx], srows, sem_s).wait()
        pltpu.async_copy(asd_sh.at[didx.at[0]], drows, sem_d).wait()

        def _edge(i, _):
            sv = srows[i]
            dv = drows[i]
            al = sv + dv.at[shift].get(mode="promise_in_bounds")
            al = jnp.where(al > 0.0, al, 0.2 * al)
            exrows[i] = jnp.exp(al)
            return _

        lax.fori_loop(0, 128, _edge, None, unroll=4)
        pltpu.sync_copy(exrows, exbuf_hbm.at[pl.ds(off, 128)])
        pltpu.sync_copy(exrows, den_sh.at[didx.at[0]], add=True)
        return _

    lax.fori_loop(0, NBATCH, _batch, None)
    plsc.subcore_barrier()

    @pl.when(c == 0)
    def _():
        pltpu.sync_copy(den_sh.at[rows], denA_hbm.at[rows])

    @pl.when(c == 1)
    def _():
        pltpu.sync_copy(den_sh.at[rows], denB_hbm.at[rows])


def _sc_phase1(asd, src, dst):
    return pl.kernel(
        _sc_phase1_body,
        out_type=[
            jax.ShapeDtypeStruct((EP, 16), jnp.float32),
            jax.ShapeDtypeStruct((NP, 16), jnp.float32),
            jax.ShapeDtypeStruct((NP, 16), jnp.float32),
        ],
        mesh=plsc.VectorSubcoreMesh(**_MESH),
        compiler_params=pltpu.CompilerParams(use_tc_tiling_on_sc=False),
        scratch_types=[
            pltpu.VMEM((128,), jnp.int32),
            pltpu.VMEM((1, 128), jnp.int32),
            pltpu.VMEM((128, 16), jnp.float32),
            pltpu.VMEM((128, 16), jnp.float32),
            pltpu.VMEM((128, 16), jnp.float32),
            pltpu.VMEM_SHARED((NP, 16), jnp.float32),
            pltpu.VMEM_SHARED((NP, 16), jnp.float32),
            pltpu.SemaphoreType.DMA,
            pltpu.SemaphoreType.DMA,
        ],
    )(asd, src, dst)


def _sc_phase2_body(vr, h_hbm, src_hbm, dst_hbm, exbuf_hbm, denA_hbm, denB_hbm,
                    outA_hbm, outB_hbm,
                    sidx, didx, hrows, exrows, drows, dstg, dstg2,
                    h_sh, den_sh, out_sh, sem_h, sem_d):
    c = lax.axis_index("c")
    s = lax.axis_index("s")
    wid = c * 16 + s
    base = wid * (EP // 32)
    rbase = s * TILE_ROWS
    rows = pl.ds(rbase, TILE_ROWS)

    pltpu.sync_copy(h_hbm.at[rows], h_sh.at[rows])

    def _dchunk(j, _):
        sl = pl.ds(rbase + j * 128, 128)
        pltpu.sync_copy(denA_hbm.at[sl], dstg)
        pltpu.sync_copy(denB_hbm.at[sl], dstg2)

        def _comb(i, __):
            dstg[i] = dstg[i] + dstg2[i]
            return __

        lax.fori_loop(0, 128, _comb, None, unroll=8)
        pltpu.sync_copy(dstg, den_sh.at[sl])
        return _

    lax.fori_loop(0, TILE_ROWS // 128, _dchunk, None)

    def _zero(i, _):
        for k in range(vr):
            hrows[i, pl.ds(k * 16, 16)] = jnp.zeros((16,), jnp.float32)
        return _

    lax.fori_loop(0, 128, _zero, None, unroll=8)

    def _zcp(j, _):
        pltpu.sync_copy(hrows, out_sh.at[pl.ds(rbase + j * 128, 128)])
        return _

    lax.fori_loop(0, TILE_ROWS // 128, _zcp, None)
    plsc.subcore_barrier()

    lane = lax.broadcasted_iota(jnp.int32, (16,), 0)
    bidx = [2 * k + (lane >> 3) for k in range(vr)]

    def _batch(b, _):
        off = base + b * 128
        pltpu.sync_copy(src_hbm.at[pl.ds(off, 128)], sidx)
        pltpu.sync_copy(dst_hbm.at[pl.ds(off, 128)], didx.at[0])
        pltpu.async_copy(h_sh.at[sidx], hrows, sem_h).wait()
        pltpu.async_copy(den_sh.at[didx.at[0]], drows, sem_d).wait()
        pltpu.sync_copy(exbuf_hbm.at[pl.ds(off, 128)], exrows)

        def _edge(i, _):
            coef = exrows[i] / drows[i]
            for k in range(vr):
                ck = coef.at[bidx[k]].get(mode="promise_in_bounds")
                hrows[i, pl.ds(k * 16, 16)] = hrows[i, pl.ds(k * 16, 16)] * ck
            return _

        lax.fori_loop(0, 128, _edge, None, unroll=2)
        pltpu.sync_copy(hrows, out_sh.at[didx.at[0]], add=True)
        return _

    lax.fori_loop(0, NBATCH, _batch, None)
    plsc.subcore_barrier()

    @pl.when(c == 0)
    def _():
        pltpu.sync_copy(out_sh.at[rows], outA_hbm.at[rows])

    @pl.when(c == 1)
    def _():
        pltpu.sync_copy(out_sh.at[rows], outB_hbm.at[rows])


def _sc_phase2(vr, h, src, dst, exbuf, denA, denB):
    d = 16 * vr
    return pl.kernel(
        functools.partial(_sc_phase2_body, vr),
        out_type=[
            jax.ShapeDtypeStruct((NP, d), jnp.float32),
            jax.ShapeDtypeStruct((NP, d), jnp.float32),
        ],
        mesh=plsc.VectorSubcoreMesh(**_MESH),
        compiler_params=pltpu.CompilerParams(use_tc_tiling_on_sc=False),
        scratch_types=[
            pltpu.VMEM((128,), jnp.int32),
            pltpu.VMEM((1, 128), jnp.int32),
            pltpu.VMEM((128, d), jnp.float32),
            pltpu.VMEM((128, 16), jnp.float32),
            pltpu.VMEM((128, 16), jnp.float32),
            pltpu.VMEM((128, 16), jnp.float32),
            pltpu.VMEM((128, 16), jnp.float32),
            pltpu.VMEM_SHARED((NP, d), jnp.float32),
            pltpu.VMEM_SHARED((NP, 16), jnp.float32),
            pltpu.VMEM_SHARED((NP, d), jnp.float32),
            pltpu.SemaphoreType.DMA,
            pltpu.SemaphoreType.DMA,
        ],
    )(h, src, dst, exbuf, denA, denB)


def kernel(x, edge_index, batch, W1, att_src1, att_dst1, b1,
           W2, att_src2, att_dst2, b2):
    f32 = jnp.float32
    x = x.astype(f32)

    x_pad = jnp.pad(x, ((0, NP - N), (0, 0)))
    ndum = EP - E
    dum_src = (jnp.arange(ndum, dtype=jnp.int32) * 37) % N
    dum_dst = N + (jnp.arange(ndum, dtype=jnp.int32) % (NP - N))
    src = jnp.concatenate([edge_index[0].astype(jnp.int32), dum_src])
    dst = jnp.concatenate([edge_index[1].astype(jnp.int32), dum_dst])

    blk = jnp.repeat(jnp.eye(H, dtype=f32), C, axis=0)
    As1 = blk * att_src1.reshape(HC)[:, None]
    Ad1 = blk * att_dst1.reshape(HC)[:, None]
    A1 = jnp.concatenate([As1, Ad1], axis=1)

    W2pad = jnp.pad(W2, ((0, 0), (0, 16 - OUT)))
    a2s = jnp.pad(att_src2.reshape(OUT), (0, 16 - OUT))
    a2d = jnp.pad(att_dst2.reshape(OUT), (0, 16 - OUT))
    A2 = jnp.concatenate([jnp.tile(a2s[:, None], (1, 8)),
                          jnp.tile(a2d[:, None], (1, 8))], axis=1)

    b1_2d = b1.reshape(1, HC)
    b2_2d = jnp.pad(b2, (0, 16 - OUT)).reshape(1, 16)
    batch3 = batch.astype(jnp.int32).reshape(N // 1000, 1, 1000)

    h1, asd1 = _k1(x_pad, W1, A1)
    ex1, den1A, den1B = _sc_phase1(asd1, src, dst)
    out1A, out1B = _sc_phase2(4, h1, src, dst, ex1, den1A, den1B)

    h2, asd2 = _k5(out1A, out1B, b1_2d, W2pad, A2)
    ex2, den2A, den2B = _sc_phase1(asd2, src, dst)
    out2A, out2B = _sc_phase2(1, h2, src, dst, ex2, den2A, den2B)

    pooled = _k8(out2A[:N], out2B[:N], batch3, b2_2d)
    return pooled[:, :OUT]


# speedup vs baseline: 58.6838x; 1.5365x over previous
import functools

import jax
import jax.numpy as jnp
from jax import lax
from jax.experimental import pallas as pl
from jax.experimental.pallas import tpu as pltpu
from jax.experimental.pallas import tpu_sc as plsc

N = 10000
E = 320000
F_IN = 128
H = 8
C = 8
HC = 64
OUT = 10
G = 128

NP = 10240
EP = 32 * NP
TILE_ROWS = NP // 16
NBATCH = (EP // 32) // 128


def _k1_body(x_ref, w_ref, a_ref, h_ref, asd_ref):
    h = jnp.dot(x_ref[...], w_ref[...], preferred_element_type=jnp.float32)
    h_ref[...] = h
    asd_ref[...] = jnp.dot(h, a_ref[...], preferred_element_type=jnp.float32)


def _k1(x_pad, W1, A1):
    B = 1280
    return pl.pallas_call(
        _k1_body,
        grid=(NP // B,),
        in_specs=[
            pl.BlockSpec((B, F_IN), lambda i: (i, 0)),
            pl.BlockSpec((F_IN, HC), lambda i: (0, 0)),
            pl.BlockSpec((HC, 16), lambda i: (0, 0)),
        ],
        out_specs=[
            pl.BlockSpec((B, HC), lambda i: (i, 0)),
            pl.BlockSpec((B, 16), lambda i: (i, 0)),
        ],
        out_shape=[
            jax.ShapeDtypeStruct((NP, HC), jnp.float32),
            jax.ShapeDtypeStruct((NP, 16), jnp.float32),
        ],
    )(x_pad, W1, A1)


def _k5_body(oa_ref, ob_ref, b1_ref, w2_ref, a2_ref, h2_ref, asd_ref):
    o = oa_ref[...] + ob_ref[...] + b1_ref[...]
    x2 = jnp.where(o > 0.0, o, jnp.exp(jnp.minimum(o, 0.0)) - 1.0)
    h2 = jnp.dot(x2, w2_ref[...], preferred_element_type=jnp.float32)
    h2_ref[...] = h2
    asd_ref[...] = jnp.dot(h2, a2_ref[...], preferred_element_type=jnp.float32)


def _k5(outA, outB, b1_2d, W2pad, A2):
    B = 1280
    return pl.pallas_call(
        _k5_body,
        grid=(NP // B,),
        in_specs=[
            pl.BlockSpec((B, HC), lambda i: (i, 0)),
            pl.BlockSpec((B, HC), lambda i: (i, 0)),
            pl.BlockSpec((1, HC), lambda i: (0, 0)),
            pl.BlockSpec((HC, 16), lambda i: (0, 0)),
            pl.BlockSpec((16, 16), lambda i: (0, 0)),
        ],
        out_specs=[
            pl.BlockSpec((B, 16), lambda i: (i, 0)),
            pl.BlockSpec((B, 16), lambda i: (i, 0)),
        ],
        out_shape=[
            jax.ShapeDtypeStruct((NP, 16), jnp.float32),
            jax.ShapeDtypeStruct((NP, 16), jnp.float32),
        ],
    )(outA, outB, b1_2d, W2pad, A2)


def _k8_body(oa_ref, ob_ref, b3_ref, b2_ref, out_ref, acc_ref, cnt_ref):
    i = pl.program_id(0)

    @pl.when(i == 0)
    def _():
        acc_ref[...] = jnp.zeros_like(acc_ref)
        cnt_ref[...] = jnp.zeros_like(cnt_ref)

    rows = oa_ref[...] + ob_ref[...]
    ids = b3_ref[0]
    gids = lax.broadcasted_iota(jnp.int32, (G, ids.shape[1]), 0)
    mask = (ids == gids).astype(jnp.float32)
    acc_ref[...] += jnp.dot(mask, rows, preferred_element_type=jnp.float32)
    cnt_ref[...] += jnp.sum(mask, axis=1, keepdims=True)

    @pl.when(i == pl.num_programs(0) - 1)
    def _():
        cnt = cnt_ref[...]
        out_ref[...] = (acc_ref[...] + cnt * b2_ref[...]) / jnp.maximum(cnt, 1.0)


def _k8(o2A, o2B, batch3, b2_2d):
    B = 1000
    return pl.pallas_call(
        _k8_body,
        grid=(N // B,),
        in_specs=[
            pl.BlockSpec((B, 16), lambda i: (i, 0)),
            pl.BlockSpec((B, 16), lambda i: (i, 0)),
            pl.BlockSpec((1, 1, B), lambda i: (i, 0, 0)),
            pl.BlockSpec((1, 16), lambda i: (0, 0)),
        ],
        out_specs=pl.BlockSpec((G, 16), lambda i: (0, 0)),
        out_shape=jax.ShapeDtypeStruct((G, 16), jnp.float32),
        scratch_shapes=[
            pltpu.VMEM((G, 16), jnp.float32),
            pltpu.VMEM((G, 1), jnp.float32),
        ],
    )(o2A, o2B, batch3, b2_2d)


_MESH = dict(core_axis_name="c", subcore_axis_name="s")


_CHB = 16
_NCH = NBATCH // _CHB


def _sc_phase1_body(asd_hbm, src_hbm, dst_hbm, exbuf_hbm, denA_hbm, denB_hbm,
                    sidxs, didxs, srows0, drows0, exrows0,
                    srows1, drows1, exrows1, asd_sh, den_sh,
                    sg0s, sg0d, sg1s, sg1d, sw0e, sw0c, sw1e, sw1c):
    c = lax.axis_index("c")
    s = lax.axis_index("s")
    wid = c * 16 + s
    base_b = wid * NBATCH
    rbase = s * TILE_ROWS
    rows = pl.ds(rbase, TILE_ROWS)

    pltpu.sync_copy(asd_hbm.at[rows], asd_sh.at[rows])

    def _zero(i, _):
        exrows0[i] = jnp.zeros((16,), jnp.float32)
        return _

    lax.fori_loop(0, 128, _zero, None, unroll=8)

    def _zcp(j, _):
        pltpu.sync_copy(exrows0, den_sh.at[pl.ds(rbase + j * 128, 128)])
        return _

    lax.fori_loop(0, TILE_ROWS // 128, _zcp, None)
    plsc.subcore_barrier()

    lane = lax.broadcasted_iota(jnp.int32, (16,), 0)
    shift = 8 + (lane % 8)

    slots = [(srows0, drows0, exrows0, sg0s, sg0d, sw0e, sw0c),
             (srows1, drows1, exrows1, sg1s, sg1d, sw1e, sw1c)]

    def _start(j, slot):
        sr, dr, _, ss, sd, _, _ = slots[slot]
        cps = pltpu.async_copy(asd_sh.at[sidxs.at[j]], sr, ss)
        cpd = pltpu.async_copy(asd_sh.at[didxs.at[j]], dr, sd)
        return cps, cpd

    def _finish(cc, j, slot, cps, cpd):
        sr, dr, ex, _, _, se, sc_ = slots[slot]
        cps.wait()
        cpd.wait()

        def _edge(i, _):
            al = sr[i] + dr[i].at[shift].get(mode="promise_in_bounds")
            al = jnp.where(al > 0.0, al, 0.2 * al)
            ex[i] = jnp.exp(al)
            return _

        lax.fori_loop(0, 128, _edge, None, unroll=4)
        off = (base_b + cc * _CHB + j) * 128
        cpe = pltpu.async_copy(ex, exbuf_hbm.at[pl.ds(off, 128)], se)
        cpc = pltpu.async_copy(ex, den_sh.at[didxs.at[j]], sc_, add=True)
        return cpe, cpc

    def _chunk(cc, _):
        pltpu.sync_copy(src_hbm.at[pl.ds(base_b + cc * _CHB, _CHB)], sidxs)
        pltpu.sync_copy(dst_hbm.at[pl.ds(base_b + cc * _CHB, _CHB)], didxs)

        def _pair(p, __):
            j0 = p * 2
            g0 = _start(j0, 0)
            g1 = _start(j0 + 1, 1)
            cpe0, cpc0 = _finish(cc, j0, 0, *g0)
            cpe1, cpc1 = _finish(cc, j0 + 1, 1, *g1)
            cpe0.wait()
            cpc0.wait()
            cpe1.wait()
            cpc1.wait()
            return __

        lax.fori_loop(0, _CHB // 2, _pair, None)
        return _

    lax.fori_loop(0, _NCH, _chunk, None)
    plsc.subcore_barrier()

    @pl.when(c == 0)
    def _():
        pltpu.sync_copy(den_sh.at[rows], denA_hbm.at[rows])

    @pl.when(c == 1)
    def _():
        pltpu.sync_copy(den_sh.at[rows], denB_hbm.at[rows])


def _sc_phase1(asd, src2, dst2):
    return pl.kernel(
        _sc_phase1_body,
        out_type=[
            jax.ShapeDtypeStruct((EP, 16), jnp.float32),
            jax.ShapeDtypeStruct((NP, 16), jnp.float32),
            jax.ShapeDtypeStruct((NP, 16), jnp.float32),
        ],
        mesh=plsc.VectorSubcoreMesh(**_MESH),
        compiler_params=pltpu.CompilerParams(use_tc_tiling_on_sc=False),
        scratch_types=[
            pltpu.VMEM((_CHB, 128), jnp.int32),
            pltpu.VMEM((_CHB, 128), jnp.int32),
            pltpu.VMEM((128, 16), jnp.float32),
            pltpu.VMEM((128, 16), jnp.float32),
            pltpu.VMEM((128, 16), jnp.float32),
            pltpu.VMEM((128, 16), jnp.float32),
            pltpu.VMEM((128, 16), jnp.float32),
            pltpu.VMEM((128, 16), jnp.float32),
            pltpu.VMEM_SHARED((NP, 16), jnp.float32),
            pltpu.VMEM_SHARED((NP, 16), jnp.float32),
        ] + [pltpu.SemaphoreType.DMA] * 8,
    )(asd, src2, dst2)


def _sc_phase2_body(vr, h_hbm, src_hbm, dst_hbm, exbuf_hbm, denA_hbm, denB_hbm,
                    outA_hbm, outB_hbm,
                    sidxs, didxs, hrows0, hrows1, drows0, drows1,
                    exrows0, exrows1, dstg, dstg2,
                    h_sh, den_sh, out_sh,
                    sg0h, sg0d, sg0e, sw0, sg1h, sg1d, sg1e, sw1):
    c = lax.axis_index("c")
    s = lax.axis_index("s")
    wid = c * 16 + s
    base_b = wid * NBATCH
    rbase = s * TILE_ROWS
    rows = pl.ds(rbase, TILE_ROWS)

    pltpu.sync_copy(h_hbm.at[rows], h_sh.at[rows])

    def _dchunk(j, _):
        sl = pl.ds(rbase + j * 128, 128)
        pltpu.sync_copy(denA_hbm.at[sl], dstg)
        pltpu.sync_copy(denB_hbm.at[sl], dstg2)

        def _comb(i, __):
            dstg[i] = dstg[i] + dstg2[i]
            return __

        lax.fori_loop(0, 128, _comb, None, unroll=8)
        pltpu.sync_copy(dstg, den_sh.at[sl])
        return _

    lax.fori_loop(0, TILE_ROWS // 128, _dchunk, None)

    def _zero(i, _):
        for k in range(vr):
            hrows0[i, pl.ds(k * 16, 16)] = jnp.zeros((16,), jnp.float32)
        return _

    lax.fori_loop(0, 128, _zero, None, unroll=8)

    def _zcp(j, _):
        pltpu.sync_copy(hrows0, out_sh.at[pl.ds(rbase + j * 128, 128)])
        return _

    lax.fori_loop(0, TILE_ROWS // 128, _zcp, None)
    plsc.subcore_barrier()

    lane = lax.broadcasted_iota(jnp.int32, (16,), 0)
    bidx = [2 * k + (lane >> 3) for k in range(vr)]

    slots = [(hrows0, drows0, exrows0, sg0h, sg0d, sg0e, sw0),
             (hrows1, drows1, exrows1, sg1h, sg1d, sg1e, sw1)]

    def _start(cc, j, slot):
        hr, dr, ex, sh_, sd, se, _ = slots[slot]
        off = (base_b + cc * _CHB + j) * 128
        cph = pltpu.async_copy(h_sh.at[sidxs.at[j]], hr, sh_)
        cpd = pltpu.async_copy(den_sh.at[didxs.at[j]], dr, sd)
        cpe = pltpu.async_copy(exbuf_hbm.at[pl.ds(off, 128)], ex, se)
        return cph, cpd, cpe

    def _finish(j, slot, cph, cpd, cpe):
        hr, dr, ex, _, _, _, sw_ = slots[slot]
        cph.wait()
        cpd.wait()
        cpe.wait()

        def _edge(i, _):
            coef = ex[i] / dr[i]
            for k in range(vr):
                ck = coef.at[bidx[k]].get(mode="promise_in_bounds")
                hr[i, pl.ds(k * 16, 16)] = hr[i, pl.ds(k * 16, 16)] * ck
            return _

        lax.fori_loop(0, 128, _edge, None, unroll=2)
        return pltpu.async_copy(hr, out_sh.at[didxs.at[j]], sw_, add=True)

    def _chunk(cc, _):
        pltpu.sync_copy(src_hbm.at[pl.ds(base_b + cc * _CHB, _CHB)], sidxs)
        pltpu.sync_copy(dst_hbm.at[pl.ds(base_b + cc * _CHB, _CHB)], didxs)

        def _pair(p, __):
            j0 = p * 2
            g0 = _start(cc, j0, 0)
            g1 = _start(cc, j0 + 1, 1)
            cp0 = _finish(j0, 0, *g0)
            cp1 = _finish(j0 + 1, 1, *g1)
            cp0.wait()
            cp1.wait()
            return __

        lax.fori_loop(0, _CHB // 2, _pair, None)
        return _

    lax.fori_loop(0, _NCH, _chunk, None)
    plsc.subcore_barrier()

    @pl.when(c == 0)
    def _():
        pltpu.sync_copy(out_sh.at[rows], outA_hbm.at[rows])

    @pl.when(c == 1)
    def _():
        pltpu.sync_copy(out_sh.at[rows], outB_hbm.at[rows])


def _sc_phase2(vr, h, src, dst, exbuf, denA, denB):
    d = 16 * vr
    return pl.kernel(
        functools.partial(_sc_phase2_body, vr),
        out_type=[
            jax.ShapeDtypeStruct((NP, d), jnp.float32),
            jax.ShapeDtypeStruct((NP, d), jnp.float32),
        ],
        mesh=plsc.VectorSubcoreMesh(**_MESH),
        compiler_params=pltpu.CompilerParams(use_tc_tiling_on_sc=False),
        scratch_types=[
            pltpu.VMEM((_CHB, 128), jnp.int32),
            pltpu.VMEM((_CHB, 128), jnp.int32),
            pltpu.VMEM((128, d), jnp.float32),
            pltpu.VMEM((128, d), jnp.float32),
            pltpu.VMEM((128, 16), jnp.float32),
            pltpu.VMEM((128, 16), jnp.float32),
            pltpu.VMEM((128, 16), jnp.float32),
            pltpu.VMEM((128, 16), jnp.float32),
            pltpu.VMEM((128, 16), jnp.float32),
            pltpu.VMEM((128, 16), jnp.float32),
            pltpu.VMEM_SHARED((NP, d), jnp.float32),
            pltpu.VMEM_SHARED((NP, 16), jnp.float32),
            pltpu.VMEM_SHARED((NP, d), jnp.float32),
        ] + [pltpu.SemaphoreType.DMA] * 8,
    )(h, src, dst, exbuf, denA, denB)


def kernel(x, edge_index, batch, W1, att_src1, att_dst1, b1,
           W2, att_src2, att_dst2, b2):
    f32 = jnp.float32
    x = x.astype(f32)

    x_pad = jnp.pad(x, ((0, NP - N), (0, 0)))
    ndum = EP - E
    dum_src = (jnp.arange(ndum, dtype=jnp.int32) * 37) % N
    dum_dst = N + (jnp.arange(ndum, dtype=jnp.int32) % (NP - N))
    src = jnp.concatenate([edge_index[0].astype(jnp.int32), dum_src])
    dst = jnp.concatenate([edge_index[1].astype(jnp.int32), dum_dst])
    src = src.reshape(EP // 128, 128)
    dst = dst.reshape(EP // 128, 128)

    blk = jnp.repeat(jnp.eye(H, dtype=f32), C, axis=0)
    As1 = blk * att_src1.reshape(HC)[:, None]
    Ad1 = blk * att_dst1.reshape(HC)[:, None]
    A1 = jnp.concatenate([As1, Ad1], axis=1)

    W2pad = jnp.pad(W2, ((0, 0), (0, 16 - OUT)))
    a2s = jnp.pad(att_src2.reshape(OUT), (0, 16 - OUT))
    a2d = jnp.pad(att_dst2.reshape(OUT), (0, 16 - OUT))
    A2 = jnp.concatenate([jnp.tile(a2s[:, None], (1, 8)),
                          jnp.tile(a2d[:, None], (1, 8))], axis=1)

    b1_2d = b1.reshape(1, HC)
    b2_2d = jnp.pad(b2, (0, 16 - OUT)).reshape(1, 16)
    batch3 = batch.astype(jnp.int32).reshape(N // 1000, 1, 1000)

    h1, asd1 = _k1(x_pad, W1, A1)
    ex1, den1A, den1B = _sc_phase1(asd1, src, dst)
    out1A, out1B = _sc_phase2(4, h1, src, dst, ex1, den1A, den1B)

    h2, asd2 = _k5(out1A, out1B, b1_2d, W2pad, A2)
    ex2, den2A, den2B = _sc_phase1(asd2, src, dst)
    out2A, out2B = _sc_phase2(1, h2, src, dst, ex2, den2A, den2B)

    pooled = _k8(out2A[:N], out2B[:N], batch3, b2_2d)
    return pooled[:, :OUT]


# speedup vs baseline: 67.9783x; 1.1584x over previous
import functools

import jax
import jax.numpy as jnp
from jax import lax
from jax.experimental import pallas as pl
from jax.experimental.pallas import tpu as pltpu
from jax.experimental.pallas import tpu_sc as plsc

N = 10000
E = 320000
F_IN = 128
H = 8
C = 8
HC = 64
OUT = 10
G = 128

NP = 10240
EP = 32 * NP
TILE_ROWS = NP // 16
NBATCH = (EP // 32) // 128


def _k1_body(x_ref, w_ref, a_ref, h_ref, asd_ref):
    h = jnp.dot(x_ref[...], w_ref[...], preferred_element_type=jnp.float32)
    h_ref[...] = h
    asd_ref[...] = jnp.dot(h, a_ref[...], preferred_element_type=jnp.float32)


def _k1(x_pad, W1, A1):
    B = 1280
    return pl.pallas_call(
        _k1_body,
        grid=(NP // B,),
        in_specs=[
            pl.BlockSpec((B, F_IN), lambda i: (i, 0)),
            pl.BlockSpec((F_IN, HC), lambda i: (0, 0)),
            pl.BlockSpec((HC, 16), lambda i: (0, 0)),
        ],
        out_specs=[
            pl.BlockSpec((B, HC), lambda i: (i, 0)),
            pl.BlockSpec((B, 16), lambda i: (i, 0)),
        ],
        out_shape=[
            jax.ShapeDtypeStruct((NP, HC), jnp.float32),
            jax.ShapeDtypeStruct((NP, 16), jnp.float32),
        ],
    )(x_pad, W1, A1)


def _k5_body(oa_ref, ob_ref, b1_ref, w2_ref, a2_ref, h2_ref, asd_ref):
    o = oa_ref[...] + ob_ref[...] + b1_ref[...]
    x2 = jnp.where(o > 0.0, o, jnp.exp(jnp.minimum(o, 0.0)) - 1.0)
    h2 = jnp.dot(x2, w2_ref[...], preferred_element_type=jnp.float32)
    h2_ref[...] = h2
    asd_ref[...] = jnp.dot(h2, a2_ref[...], preferred_element_type=jnp.float32)


def _k5(outA, outB, b1_2d, W2pad, A2):
    B = 1280
    return pl.pallas_call(
        _k5_body,
        grid=(NP // B,),
        in_specs=[
            pl.BlockSpec((B, HC), lambda i: (i, 0)),
            pl.BlockSpec((B, HC), lambda i: (i, 0)),
            pl.BlockSpec((1, HC), lambda i: (0, 0)),
            pl.BlockSpec((HC, 16), lambda i: (0, 0)),
            pl.BlockSpec((16, 16), lambda i: (0, 0)),
        ],
        out_specs=[
            pl.BlockSpec((B, 16), lambda i: (i, 0)),
            pl.BlockSpec((B, 16), lambda i: (i, 0)),
        ],
        out_shape=[
            jax.ShapeDtypeStruct((NP, 16), jnp.float32),
            jax.ShapeDtypeStruct((NP, 16), jnp.float32),
        ],
    )(outA, outB, b1_2d, W2pad, A2)


def _k8_body(oa_ref, ob_ref, b3_ref, b2_ref, out_ref, acc_ref, cnt_ref):
    i = pl.program_id(0)

    @pl.when(i == 0)
    def _():
        acc_ref[...] = jnp.zeros_like(acc_ref)
        cnt_ref[...] = jnp.zeros_like(cnt_ref)

    rows = oa_ref[...] + ob_ref[...]
    ids = b3_ref[0]
    gids = lax.broadcasted_iota(jnp.int32, (G, ids.shape[1]), 0)
    mask = (ids == gids).astype(jnp.float32)
    acc_ref[...] += jnp.dot(mask, rows, preferred_element_type=jnp.float32)
    cnt_ref[...] += jnp.sum(mask, axis=1, keepdims=True)

    @pl.when(i == pl.num_programs(0) - 1)
    def _():
        cnt = cnt_ref[...]
        out_ref[...] = (acc_ref[...] + cnt * b2_ref[...]) / jnp.maximum(cnt, 1.0)


def _k8(o2A, o2B, batch3, b2_2d):
    B = 1000
    return pl.pallas_call(
        _k8_body,
        grid=(N // B,),
        in_specs=[
            pl.BlockSpec((B, 16), lambda i: (i, 0)),
            pl.BlockSpec((B, 16), lambda i: (i, 0)),
            pl.BlockSpec((1, 1, B), lambda i: (i, 0, 0)),
            pl.BlockSpec((1, 16), lambda i: (0, 0)),
        ],
        out_specs=pl.BlockSpec((G, 16), lambda i: (0, 0)),
        out_shape=jax.ShapeDtypeStruct((G, 16), jnp.float32),
        scratch_shapes=[
            pltpu.VMEM((G, 16), jnp.float32),
            pltpu.VMEM((G, 1), jnp.float32),
        ],
    )(o2A, o2B, batch3, b2_2d)


_MESH = dict(core_axis_name="c", subcore_axis_name="s")


_CHB = 16
_NCH = NBATCH // _CHB


def _sc_phase1_body(asd_hbm, src_hbm, dst_hbm, exbuf_hbm, denA_hbm, denB_hbm,
                    sidxs, didxs, srows0, drows0, exrows0,
                    srows1, drows1, exrows1, asd_sh, den_sh,
                    sg0s, sg0d, sg1s, sg1d, sw0e, sw0c, sw1e, sw1c):
    c = lax.axis_index("c")
    s = lax.axis_index("s")
    wid = c * 16 + s
    base_b = wid * NBATCH
    rbase = s * TILE_ROWS
    rows = pl.ds(rbase, TILE_ROWS)

    pltpu.sync_copy(asd_hbm.at[rows], asd_sh.at[rows])

    def _zero(i, _):
        exrows0[i] = jnp.zeros((16,), jnp.float32)
        return _

    lax.fori_loop(0, 128, _zero, None, unroll=8)

    def _zcp(j, _):
        pltpu.sync_copy(exrows0, den_sh.at[pl.ds(rbase + j * 128, 128)])
        return _

    lax.fori_loop(0, TILE_ROWS // 128, _zcp, None)
    plsc.subcore_barrier()

    lane = lax.broadcasted_iota(jnp.int32, (16,), 0)
    shift = 8 + (lane % 8)

    slots = [(srows0, drows0, exrows0, sg0s, sg0d, sw0e, sw0c),
             (srows1, drows1, exrows1, sg1s, sg1d, sw1e, sw1c)]

    def _start(j, slot):
        sr, dr, _, ss, sd, _, _ = slots[slot]
        cps = pltpu.async_copy(asd_sh.at[sidxs.at[j]], sr, ss)
        cpd = pltpu.async_copy(asd_sh.at[didxs.at[j]], dr, sd)
        return cps, cpd

    def _finish(cc, j, slot, cps, cpd):
        sr, dr, ex, _, _, se, sc_ = slots[slot]
        cps.wait()
        cpd.wait()

        def _edge(i, _):
            al = sr[i] + dr[i].at[shift].get(mode="promise_in_bounds")
            ex[i] = jnp.exp(jnp.maximum(al, 0.2 * al))
            return _

        lax.fori_loop(0, 128, _edge, None, unroll=4)
        off = (base_b + cc * _CHB + j) * 128
        cpe = pltpu.async_copy(ex, exbuf_hbm.at[pl.ds(off, 128)], se)
        cpc = pltpu.async_copy(ex, den_sh.at[didxs.at[j]], sc_, add=True)
        return cpe, cpc

    def _chunk(cc, _):
        pltpu.sync_copy(src_hbm.at[pl.ds(base_b + cc * _CHB, _CHB)], sidxs)
        pltpu.sync_copy(dst_hbm.at[pl.ds(base_b + cc * _CHB, _CHB)], didxs)

        def _pair(p, __):
            j0 = p * 2
            g0 = _start(j0, 0)
            g1 = _start(j0 + 1, 1)
            cpe0, cpc0 = _finish(cc, j0, 0, *g0)
            cpe1, cpc1 = _finish(cc, j0 + 1, 1, *g1)
            cpe0.wait()
            cpc0.wait()
            cpe1.wait()
            cpc1.wait()
            return __

        lax.fori_loop(0, _CHB // 2, _pair, None)
        return _

    lax.fori_loop(0, _NCH, _chunk, None)
    plsc.subcore_barrier()

    @pl.when(c == 0)
    def _():
        pltpu.sync_copy(den_sh.at[rows], denA_hbm.at[rows])

    @pl.when(c == 1)
    def _():
        pltpu.sync_copy(den_sh.at[rows], denB_hbm.at[rows])


def _sc_phase1(asd, src2, dst2):
    return pl.kernel(
        _sc_phase1_body,
        out_type=[
            jax.ShapeDtypeStruct((EP, 16), jnp.float32),
            jax.ShapeDtypeStruct((NP, 16), jnp.float32),
            jax.ShapeDtypeStruct((NP, 16), jnp.float32),
        ],
        mesh=plsc.VectorSubcoreMesh(**_MESH),
        compiler_params=pltpu.CompilerParams(use_tc_tiling_on_sc=False),
        scratch_types=[
            pltpu.VMEM((_CHB, 128), jnp.int32),
            pltpu.VMEM((_CHB, 128), jnp.int32),
            pltpu.VMEM((128, 16), jnp.float32),
            pltpu.VMEM((128, 16), jnp.float32),
            pltpu.VMEM((128, 16), jnp.float32),
            pltpu.VMEM((128, 16), jnp.float32),
            pltpu.VMEM((128, 16), jnp.float32),
            pltpu.VMEM((128, 16), jnp.float32),
            pltpu.VMEM_SHARED((NP, 16), jnp.float32),
            pltpu.VMEM_SHARED((NP, 16), jnp.float32),
        ] + [pltpu.SemaphoreType.DMA] * 8,
    )(asd, src2, dst2)


def _sc_phase2_body(vr, h_hbm, src_hbm, dst_hbm, exbuf_hbm, denA_hbm, denB_hbm,
                    outA_hbm, outB_hbm,
                    sidxs, didxs, hrows0, hrows1,
                    exrows0, exrows1, dstg, dstg2,
                    h_sh, out_sh,
                    sg0h, sg0e, sw0, sg1h, sg1e, sw1):
    c = lax.axis_index("c")
    s = lax.axis_index("s")
    wid = c * 16 + s
    base_b = wid * NBATCH
    rbase = s * TILE_ROWS
    rows = pl.ds(rbase, TILE_ROWS)

    pltpu.sync_copy(h_hbm.at[rows], h_sh.at[rows])

    def _zero(i, _):
        for k in range(vr):
            hrows0[i, pl.ds(k * 16, 16)] = jnp.zeros((16,), jnp.float32)
        return _

    lax.fori_loop(0, 128, _zero, None, unroll=8)

    def _zcp(j, _):
        pltpu.sync_copy(hrows0, out_sh.at[pl.ds(rbase + j * 128, 128)])
        return _

    lax.fori_loop(0, TILE_ROWS // 128, _zcp, None)
    plsc.subcore_barrier()

    lane = lax.broadcasted_iota(jnp.int32, (16,), 0)
    bidx = [2 * k + (lane >> 3) for k in range(vr)]

    slots = [(hrows0, exrows0, sg0h, sg0e, sw0),
             (hrows1, exrows1, sg1h, sg1e, sw1)]

    def _start(cc, j, slot):
        hr, ex, sh_, se, _ = slots[slot]
        off = (base_b + cc * _CHB + j) * 128
        cph = pltpu.async_copy(h_sh.at[sidxs.at[j]], hr, sh_)
        cpe = pltpu.async_copy(exbuf_hbm.at[pl.ds(off, 128)], ex, se)
        return cph, cpe

    def _finish(j, slot, cph, cpe):
        hr, ex, _, _, sw_ = slots[slot]
        cph.wait()
        cpe.wait()

        def _edge(i, _):
            exr = ex[i]
            for k in range(vr):
                ck = exr.at[bidx[k]].get(mode="promise_in_bounds")
                hr[i, pl.ds(k * 16, 16)] = hr[i, pl.ds(k * 16, 16)] * ck
            return _

        lax.fori_loop(0, 128, _edge, None, unroll=2)
        return pltpu.async_copy(hr, out_sh.at[didxs.at[j]], sw_, add=True)

    def _chunk(cc, _):
        pltpu.sync_copy(src_hbm.at[pl.ds(base_b + cc * _CHB, _CHB)], sidxs)
        pltpu.sync_copy(dst_hbm.at[pl.ds(base_b + cc * _CHB, _CHB)], didxs)

        def _pair(p, __):
            j0 = p * 2
            g0 = _start(cc, j0, 0)
            g1 = _start(cc, j0 + 1, 1)
            cp0 = _finish(j0, 0, *g0)
            cp1 = _finish(j0 + 1, 1, *g1)
            cp0.wait()
            cp1.wait()
            return __

        lax.fori_loop(0, _CHB // 2, _pair, None)
        return _

    lax.fori_loop(0, _NCH, _chunk, None)
    plsc.subcore_barrier()

    def _dump(jc, _):
        sl = pl.ds(rbase + jc * 128, 128)
        pltpu.sync_copy(denA_hbm.at[sl], dstg)
        pltpu.sync_copy(denB_hbm.at[sl], dstg2)
        pltpu.sync_copy(out_sh.at[sl], hrows0)

        def _row(i, __):
            den = dstg[i] + dstg2[i]
            for k in range(vr):
                dk = den.at[bidx[k]].get(mode="promise_in_bounds")
                dk = jnp.maximum(dk, 1e-30)
                hr = hrows0[i, pl.ds(k * 16, 16)]
                hrows0[i, pl.ds(k * 16, 16)] = hr / dk
            return __

        lax.fori_loop(0, 128, _row, None, unroll=2)

        @pl.when(c == 0)
        def _():
            pltpu.sync_copy(hrows0, outA_hbm.at[sl])

        @pl.when(c == 1)
        def _():
            pltpu.sync_copy(hrows0, outB_hbm.at[sl])

        return _

    lax.fori_loop(0, TILE_ROWS // 128, _dump, None)


def _sc_phase2(vr, h, src, dst, exbuf, denA, denB):
    d = 16 * vr
    return pl.kernel(
        functools.partial(_sc_phase2_body, vr),
        out_type=[
            jax.ShapeDtypeStruct((NP, d), jnp.float32),
            jax.ShapeDtypeStruct((NP, d), jnp.float32),
        ],
        mesh=plsc.VectorSubcoreMesh(**_MESH),
        compiler_params=pltpu.CompilerParams(use_tc_tiling_on_sc=False),
        scratch_types=[
            pltpu.VMEM((_CHB, 128), jnp.int32),
            pltpu.VMEM((_CHB, 128), jnp.int32),
            pltpu.VMEM((128, d), jnp.float32),
            pltpu.VMEM((128, d), jnp.float32),
            pltpu.VMEM((128, 16), jnp.float32),
            pltpu.VMEM((128, 16), jnp.float32),
            pltpu.VMEM((128, 16), jnp.float32),
            pltpu.VMEM((128, 16), jnp.float32),
            pltpu.VMEM_SHARED((NP, d), jnp.float32),
            pltpu.VMEM_SHARED((NP, d), jnp.float32),
        ] + [pltpu.SemaphoreType.DMA] * 6,
    )(h, src, dst, exbuf, denA, denB)


def kernel(x, edge_index, batch, W1, att_src1, att_dst1, b1,
           W2, att_src2, att_dst2, b2):
    f32 = jnp.float32
    x = x.astype(f32)

    x_pad = jnp.pad(x, ((0, NP - N), (0, 0)))
    ndum = EP - E
    dum_src = (jnp.arange(ndum, dtype=jnp.int32) * 37) % N
    dum_dst = N + (jnp.arange(ndum, dtype=jnp.int32) % (NP - N))
    src = jnp.concatenate([edge_index[0].astype(jnp.int32), dum_src])
    dst = jnp.concatenate([edge_index[1].astype(jnp.int32), dum_dst])
    src = src.reshape(EP // 128, 128)
    dst = dst.reshape(EP // 128, 128)

    blk = jnp.repeat(jnp.eye(H, dtype=f32), C, axis=0)
    As1 = blk * att_src1.reshape(HC)[:, None]
    Ad1 = blk * att_dst1.reshape(HC)[:, None]
    A1 = jnp.concatenate([As1, Ad1], axis=1)

    W2pad = jnp.pad(W2, ((0, 0), (0, 16 - OUT)))
    a2s = jnp.pad(att_src2.reshape(OUT), (0, 16 - OUT))
    a2d = jnp.pad(att_dst2.reshape(OUT), (0, 16 - OUT))
    A2 = jnp.concatenate([jnp.tile(a2s[:, None], (1, 8)),
                          jnp.tile(a2d[:, None], (1, 8))], axis=1)

    b1_2d = b1.reshape(1, HC)
    b2_2d = jnp.pad(b2, (0, 16 - OUT)).reshape(1, 16)
    batch3 = batch.astype(jnp.int32).reshape(N // 1000, 1, 1000)

    h1, asd1 = _k1(x_pad, W1, A1)
    ex1, den1A, den1B = _sc_phase1(asd1, src, dst)
    out1A, out1B = _sc_phase2(4, h1, src, dst, ex1, den1A, den1B)

    h2, asd2 = _k5(out1A, out1B, b1_2d, W2pad, A2)
    ex2, den2A, den2B = _sc_phase1(asd2, src, dst)
    out2A, out2B = _sc_phase2(1, h2, src, dst, ex2, den2A, den2B)

    pooled = _k8(out2A[:N], out2B[:N], batch3, b2_2d)
    return pooled[:, :OUT]


# speedup vs baseline: 71.2990x; 1.0488x over previous
import functools

import jax
import jax.numpy as jnp
from jax import lax
from jax.experimental import pallas as pl
from jax.experimental.pallas import tpu as pltpu
from jax.experimental.pallas import tpu_sc as plsc

N = 10000
E = 320000
F_IN = 128
H = 8
C = 8
HC = 64
OUT = 10
G = 128

NP = 10240
EP = 32 * NP
TILE_ROWS = NP // 16
NBATCH = (EP // 32) // 128


def _k1_body(x_ref, w_ref, a_ref, h_ref, asd_ref):
    h = jnp.dot(x_ref[...], w_ref[...], preferred_element_type=jnp.float32)
    h_ref[...] = h
    asd_ref[...] = jnp.dot(h, a_ref[...], preferred_element_type=jnp.float32)


def _k1(x_pad, W1, A1):
    B = 1280
    return pl.pallas_call(
        _k1_body,
        grid=(NP // B,),
        in_specs=[
            pl.BlockSpec((B, F_IN), lambda i: (i, 0)),
            pl.BlockSpec((F_IN, HC), lambda i: (0, 0)),
            pl.BlockSpec((HC, 16), lambda i: (0, 0)),
        ],
        out_specs=[
            pl.BlockSpec((B, HC), lambda i: (i, 0)),
            pl.BlockSpec((B, 16), lambda i: (i, 0)),
        ],
        out_shape=[
            jax.ShapeDtypeStruct((NP, HC), jnp.float32),
            jax.ShapeDtypeStruct((NP, 16), jnp.float32),
        ],
    )(x_pad, W1, A1)


def _k5_body(oa_ref, ob_ref, b1_ref, w2_ref, a2_ref, h2_ref, asd_ref):
    o = oa_ref[...] + ob_ref[...] + b1_ref[...]
    x2 = jnp.where(o > 0.0, o, jnp.exp(jnp.minimum(o, 0.0)) - 1.0)
    h2 = jnp.dot(x2, w2_ref[...], preferred_element_type=jnp.float32)
    h2_ref[...] = h2
    asd_ref[...] = jnp.dot(h2, a2_ref[...], preferred_element_type=jnp.float32)


def _k5(outA, outB, b1_2d, W2pad, A2):
    B = 1280
    return pl.pallas_call(
        _k5_body,
        grid=(NP // B,),
        in_specs=[
            pl.BlockSpec((B, HC), lambda i: (i, 0)),
            pl.BlockSpec((B, HC), lambda i: (i, 0)),
            pl.BlockSpec((1, HC), lambda i: (0, 0)),
            pl.BlockSpec((HC, 16), lambda i: (0, 0)),
            pl.BlockSpec((16, 16), lambda i: (0, 0)),
        ],
        out_specs=[
            pl.BlockSpec((B, 16), lambda i: (i, 0)),
            pl.BlockSpec((B, 16), lambda i: (i, 0)),
        ],
        out_shape=[
            jax.ShapeDtypeStruct((NP, 16), jnp.float32),
            jax.ShapeDtypeStruct((NP, 16), jnp.float32),
        ],
    )(outA, outB, b1_2d, W2pad, A2)


def _k8_body(oa_ref, ob_ref, b3_ref, b2_ref, out_ref, acc_ref, cnt_ref):
    i = pl.program_id(0)

    @pl.when(i == 0)
    def _():
        acc_ref[...] = jnp.zeros_like(acc_ref)
        cnt_ref[...] = jnp.zeros_like(cnt_ref)

    rows = oa_ref[...] + ob_ref[...]
    ids = b3_ref[0]
    gids = lax.broadcasted_iota(jnp.int32, (G, ids.shape[1]), 0)
    mask = (ids == gids).astype(jnp.float32)
    acc_ref[...] += jnp.dot(mask, rows, preferred_element_type=jnp.float32)
    cnt_ref[...] += jnp.sum(mask, axis=1, keepdims=True)

    @pl.when(i == pl.num_programs(0) - 1)
    def _():
        cnt = cnt_ref[...]
        out_ref[...] = (acc_ref[...] + cnt * b2_ref[...]) / jnp.maximum(cnt, 1.0)


def _k8(o2A, o2B, batch3, b2_2d):
    B = 1000
    return pl.pallas_call(
        _k8_body,
        grid=(N // B,),
        in_specs=[
            pl.BlockSpec((B, 16), lambda i: (i, 0)),
            pl.BlockSpec((B, 16), lambda i: (i, 0)),
            pl.BlockSpec((1, 1, B), lambda i: (i, 0, 0)),
            pl.BlockSpec((1, 16), lambda i: (0, 0)),
        ],
        out_specs=pl.BlockSpec((G, 16), lambda i: (0, 0)),
        out_shape=jax.ShapeDtypeStruct((G, 16), jnp.float32),
        scratch_shapes=[
            pltpu.VMEM((G, 16), jnp.float32),
            pltpu.VMEM((G, 1), jnp.float32),
        ],
    )(o2A, o2B, batch3, b2_2d)


_MESH = dict(core_axis_name="c", subcore_axis_name="s")


_CHB = 8
_NCH = NBATCH // _CHB
_NSLOT = 4


def _sc_phase1_body(asd_hbm, src_hbm, dst_hbm, exbuf_hbm, denA_hbm, denB_hbm,
                    sidxs, didxs, *rest):
    bufs = rest[:3 * _NSLOT]
    asd_sh, den_sh = rest[3 * _NSLOT:3 * _NSLOT + 2]
    sems = rest[3 * _NSLOT + 2:]
    c = lax.axis_index("c")
    s = lax.axis_index("s")
    wid = c * 16 + s
    base_b = wid * NBATCH
    rbase = s * TILE_ROWS
    rows = pl.ds(rbase, TILE_ROWS)

    slots = [(bufs[3 * t], bufs[3 * t + 1], bufs[3 * t + 2],
              sems[4 * t], sems[4 * t + 1], sems[4 * t + 2], sems[4 * t + 3])
             for t in range(_NSLOT)]

    pltpu.sync_copy(asd_hbm.at[rows], asd_sh.at[rows])
    ex0 = slots[0][2]

    def _zero(i, _):
        ex0[i] = jnp.zeros((16,), jnp.float32)
        return _

    lax.fori_loop(0, 128, _zero, None, unroll=8)

    def _zcp(j, _):
        pltpu.sync_copy(ex0, den_sh.at[pl.ds(rbase + j * 128, 128)])
        return _

    lax.fori_loop(0, TILE_ROWS // 128, _zcp, None)
    plsc.subcore_barrier()

    lane = lax.broadcasted_iota(jnp.int32, (16,), 0)
    shift = 8 + (lane % 8)

    def _start(j, slot):
        sr, dr, _, ss, sd, _, _ = slots[slot]
        cps = pltpu.async_copy(asd_sh.at[sidxs.at[j]], sr, ss)
        cpd = pltpu.async_copy(asd_sh.at[didxs.at[j]], dr, sd)
        return cps, cpd

    def _finish(cc, j, slot, cps, cpd):
        sr, dr, ex, _, _, se, sc_ = slots[slot]
        cps.wait()
        cpd.wait()

        def _edge(i, _):
            al = sr[i] + dr[i].at[shift].get(mode="promise_in_bounds")
            ex[i] = jnp.exp(jnp.maximum(al, 0.2 * al))
            return _

        lax.fori_loop(0, 128, _edge, None, unroll=8)
        off = (base_b + cc * _CHB + j) * 128
        cpe = pltpu.async_copy(ex, exbuf_hbm.at[pl.ds(off, 128)], se)
        cpc = pltpu.async_copy(ex, den_sh.at[didxs.at[j]], sc_, add=True)
        return cpe, cpc

    def _chunk(cc, _):
        pltpu.sync_copy(src_hbm.at[pl.ds(base_b + cc * _CHB, _CHB)], sidxs)
        pltpu.sync_copy(dst_hbm.at[pl.ds(base_b + cc * _CHB, _CHB)], didxs)

        def _quad(q, __):
            j0 = q * _NSLOT
            gs = [_start(j0 + t, t) for t in range(_NSLOT)]
            ws = [_finish(cc, j0 + t, t, *gs[t]) for t in range(_NSLOT)]
            for cpe, cpc in ws:
                cpe.wait()
                cpc.wait()
            return __

        lax.fori_loop(0, _CHB // _NSLOT, _quad, None)
        return _

    lax.fori_loop(0, _NCH, _chunk, None)
    plsc.subcore_barrier()

    @pl.when(c == 0)
    def _():
        pltpu.sync_copy(den_sh.at[rows], denA_hbm.at[rows])

    @pl.when(c == 1)
    def _():
        pltpu.sync_copy(den_sh.at[rows], denB_hbm.at[rows])


def _sc_phase1(asd, src2, dst2):
    return pl.kernel(
        _sc_phase1_body,
        out_type=[
            jax.ShapeDtypeStruct((EP, 16), jnp.float32),
            jax.ShapeDtypeStruct((NP, 16), jnp.float32),
            jax.ShapeDtypeStruct((NP, 16), jnp.float32),
        ],
        mesh=plsc.VectorSubcoreMesh(**_MESH),
        compiler_params=pltpu.CompilerParams(use_tc_tiling_on_sc=False),
        scratch_types=[
            pltpu.VMEM((_CHB, 128), jnp.int32),
            pltpu.VMEM((_CHB, 128), jnp.int32),
        ] + [pltpu.VMEM((128, 16), jnp.float32)] * (3 * _NSLOT) + [
            pltpu.VMEM_SHARED((NP, 16), jnp.float32),
            pltpu.VMEM_SHARED((NP, 16), jnp.float32),
        ] + [pltpu.SemaphoreType.DMA] * (4 * _NSLOT),
    )(asd, src2, dst2)


def _sc_phase2_body(vr, h_hbm, src_hbm, dst_hbm, exbuf_hbm, denA_hbm, denB_hbm,
                    outA_hbm, outB_hbm,
                    sidxs, didxs, *rest):
    bufs = rest[:2 * _NSLOT]
    dstg, dstg2, h_sh, out_sh = rest[2 * _NSLOT:2 * _NSLOT + 4]
    sems = rest[2 * _NSLOT + 4:]
    c = lax.axis_index("c")
    s = lax.axis_index("s")
    wid = c * 16 + s
    base_b = wid * NBATCH
    rbase = s * TILE_ROWS
    rows = pl.ds(rbase, TILE_ROWS)

    slots = [(bufs[2 * t], bufs[2 * t + 1],
              sems[3 * t], sems[3 * t + 1], sems[3 * t + 2])
             for t in range(_NSLOT)]
    hrows0 = slots[0][0]

    pltpu.sync_copy(h_hbm.at[rows], h_sh.at[rows])

    def _zero(i, _):
        for k in range(vr):
            hrows0[i, pl.ds(k * 16, 16)] = jnp.zeros((16,), jnp.float32)
        return _

    lax.fori_loop(0, 128, _zero, None, unroll=8)

    def _zcp(j, _):
        pltpu.sync_copy(hrows0, out_sh.at[pl.ds(rbase + j * 128, 128)])
        return _

    lax.fori_loop(0, TILE_ROWS // 128, _zcp, None)
    plsc.subcore_barrier()

    lane = lax.broadcasted_iota(jnp.int32, (16,), 0)
    bidx = [2 * k + (lane >> 3) for k in range(vr)]

    def _start(cc, j, slot):
        hr, ex, sh_, se, _ = slots[slot]
        off = (base_b + cc * _CHB + j) * 128
        cph = pltpu.async_copy(h_sh.at[sidxs.at[j]], hr, sh_)
        cpe = pltpu.async_copy(exbuf_hbm.at[pl.ds(off, 128)], ex, se)
        return cph, cpe

    def _finish(j, slot, cph, cpe):
        hr, ex, _, _, sw_ = slots[slot]
        cph.wait()
        cpe.wait()

        def _edge(i, _):
            exr = ex[i]
            for k in range(vr):
                ck = exr.at[bidx[k]].get(mode="promise_in_bounds")
                hr[i, pl.ds(k * 16, 16)] = hr[i, pl.ds(k * 16, 16)] * ck
            return _

        lax.fori_loop(0, 128, _edge, None, unroll=8)
        return pltpu.async_copy(hr, out_sh.at[didxs.at[j]], sw_, add=True)

    def _chunk(cc, _):
        pltpu.sync_copy(src_hbm.at[pl.ds(base_b + cc * _CHB, _CHB)], sidxs)
        pltpu.sync_copy(dst_hbm.at[pl.ds(base_b + cc * _CHB, _CHB)], didxs)

        def _quad(q, __):
            j0 = q * _NSLOT
            gs = [_start(cc, j0 + t, t) for t in range(_NSLOT)]
            ws = [_finish(j0 + t, t, *gs[t]) for t in range(_NSLOT)]
            for cp in ws:
                cp.wait()
            return __

        lax.fori_loop(0, _CHB // _NSLOT, _quad, None)
        return _

    lax.fori_loop(0, _NCH, _chunk, None)
    plsc.subcore_barrier()

    def _dump(jc, _):
        sl = pl.ds(rbase + jc * 128, 128)
        pltpu.sync_copy(denA_hbm.at[sl], dstg)
        pltpu.sync_copy(denB_hbm.at[sl], dstg2)
        pltpu.sync_copy(out_sh.at[sl], hrows0)

        def _row(i, __):
            den = dstg[i] + dstg2[i]
            for k in range(vr):
                dk = den.at[bidx[k]].get(mode="promise_in_bounds")
                dk = jnp.maximum(dk, 1e-30)
                hr = hrows0[i, pl.ds(k * 16, 16)]
                hrows0[i, pl.ds(k * 16, 16)] = hr / dk
            return __

        lax.fori_loop(0, 128, _row, None, unroll=2)

        @pl.when(c == 0)
        def _():
            pltpu.sync_copy(hrows0, outA_hbm.at[sl])

        @pl.when(c == 1)
        def _():
            pltpu.sync_copy(hrows0, outB_hbm.at[sl])

        return _

    lax.fori_loop(0, TILE_ROWS // 128, _dump, None)


def _sc_phase2(vr, h, src, dst, exbuf, denA, denB):
    d = 16 * vr
    return pl.kernel(
        functools.partial(_sc_phase2_body, vr),
        out_type=[
            jax.ShapeDtypeStruct((NP, d), jnp.float32),
            jax.ShapeDtypeStruct((NP, d), jnp.float32),
        ],
        mesh=plsc.VectorSubcoreMesh(**_MESH),
        compiler_params=pltpu.CompilerParams(use_tc_tiling_on_sc=False),
        scratch_types=[
            pltpu.VMEM((_CHB, 128), jnp.int32),
            pltpu.VMEM((_CHB, 128), jnp.int32),
        ] + [pltpu.VMEM((128, d), jnp.float32),
             pltpu.VMEM((128, 16), jnp.float32)] * _NSLOT + [
            pltpu.VMEM((128, 16), jnp.float32),
            pltpu.VMEM((128, 16), jnp.float32),
            pltpu.VMEM_SHARED((NP, d), jnp.float32),
            pltpu.VMEM_SHARED((NP, d), jnp.float32),
        ] + [pltpu.SemaphoreType.DMA] * (3 * _NSLOT),
    )(h, src, dst, exbuf, denA, denB)


def kernel(x, edge_index, batch, W1, att_src1, att_dst1, b1,
           W2, att_src2, att_dst2, b2):
    f32 = jnp.float32
    x = x.astype(f32)

    x_pad = jnp.pad(x, ((0, NP - N), (0, 0)))
    ndum = EP - E
    dum_src = (jnp.arange(ndum, dtype=jnp.int32) * 37) % N
    dum_dst = N + (jnp.arange(ndum, dtype=jnp.int32) % (NP - N))
    src = jnp.concatenate([edge_index[0].astype(jnp.int32), dum_src])
    dst = jnp.concatenate([edge_index[1].astype(jnp.int32), dum_dst])
    src = src.reshape(EP // 128, 128)
    dst = dst.reshape(EP // 128, 128)

    blk = jnp.repeat(jnp.eye(H, dtype=f32), C, axis=0)
    As1 = blk * att_src1.reshape(HC)[:, None]
    Ad1 = blk * att_dst1.reshape(HC)[:, None]
    A1 = jnp.concatenate([As1, Ad1], axis=1)

    W2pad = jnp.pad(W2, ((0, 0), (0, 16 - OUT)))
    a2s = jnp.pad(att_src2.reshape(OUT), (0, 16 - OUT))
    a2d = jnp.pad(att_dst2.reshape(OUT), (0, 16 - OUT))
    A2 = jnp.concatenate([jnp.tile(a2s[:, None], (1, 8)),
                          jnp.tile(a2d[:, None], (1, 8))], axis=1)

    b1_2d = b1.reshape(1, HC)
    b2_2d = jnp.pad(b2, (0, 16 - OUT)).reshape(1, 16)
    batch3 = batch.astype(jnp.int32).reshape(N // 1000, 1, 1000)

    h1, asd1 = _k1(x_pad, W1, A1)
    ex1, den1A, den1B = _sc_phase1(asd1, src, dst)
    out1A, out1B = _sc_phase2(4, h1, src, dst, ex1, den1A, den1B)

    h2, asd2 = _k5(out1A, out1B, b1_2d, W2pad, A2)
    ex2, den2A, den2B = _sc_phase1(asd2, src, dst)
    out2A, out2B = _sc_phase2(1, h2, src, dst, ex2, den2A, den2B)

    pooled = _k8(out2A[:N], out2B[:N], batch3, b2_2d)
    return pooled[:, :OUT]
